# baseline (speedup 1.0000x reference)
import functools
import jax
import jax.numpy as jnp
from jax import lax
from jax.experimental import pallas as pl
from jax.experimental.pallas import tpu as pltpu
from jax.experimental.pallas import tpu_sc as plsc

NC = 2
NS = 16
NW = NC * NS
CH = 128


@functools.partial(jax.jit, static_argnames=("nchunk",))
def _sc_aggregate(h, c2p, u2p, *, nchunk):
    N, D = h.shape
    acc_rows = ((N + NS * CH - 1) // (NS * CH)) * NS * CH
    zrows = acc_rows // NS

    assert nchunk % 2 == 0
    hs = nchunk // 2

    mesh = plsc.VectorSubcoreMesh(core_axis_name="c", subcore_axis_name="s")

    @functools.partial(
        pl.kernel,
        out_type=jax.ShapeDtypeStruct((NC, acc_rows, D), jnp.float32),
        mesh=mesh,
        scratch_types=[
            pltpu.VMEM((hs, CH), jnp.int32),
            pltpu.VMEM((hs, CH), jnp.int32),
            pltpu.VMEM((CH, D), jnp.float32),
            pltpu.VMEM((CH, D), jnp.float32),
            pltpu.VMEM_SHARED((acc_rows, D), jnp.float32),
            pltpu.SemaphoreType.DMA,
            pltpu.SemaphoreType.DMA,
        ],
    )
    def k(h_hbm, c2_hbm, u2_hbm, out_hbm, c2_v, u2_v, rows0_v, rows1_v,
          acc_s, gsem0, gsem1):
        rows_v = rows0_v
        cid = lax.axis_index("c")
        sid = lax.axis_index("s")
        wid = sid * NC + cid

        zeros16 = jnp.zeros((16,), jnp.float32)

        def zfill(i, carry):
            rows_v[i // (D // 16), pl.ds((i % (D // 16)) * 16, 16)] = zeros16
            return carry

        lax.fori_loop(0, CH * D // 16, zfill, 0)

        def zcopy(j, carry):
            pltpu.sync_copy(rows_v, acc_s.at[pl.ds(sid * zrows + j * CH, CH)])
            return carry

        lax.fori_loop(0, zrows // CH, zcopy, 0)
        plsc.subcore_barrier()

        for p in range(2):
            pltpu.sync_copy(c2_hbm.at[wid, pl.ds(p * hs, hs)], c2_v)
            pltpu.sync_copy(u2_hbm.at[wid, pl.ds(p * hs, hs)], u2_v)

            def chunk(j, carry):
                pltpu.async_copy(h_hbm.at[c2_v.at[j]], rows0_v, gsem0).wait()
                pltpu.sync_copy(rows0_v, acc_s.at[u2_v.at[j]], add=True)
                return carry

            lax.fori_loop(0, hs, chunk, 0)
        plsc.subcore_barrier()

        pltpu.sync_copy(
            acc_s.at[pl.ds(sid * zrows, zrows)],
            out_hbm.at[cid, pl.ds(sid * zrows, zrows)],
        )

    return k(h, c2p, u2p)


def _encoder(x, We, be, bn):
    N, D = x.shape

    def body(x_ref, w_ref, b_ref, o_ref):
        o_ref[...] = (
            jnp.dot(x_ref[...], w_ref[...], preferred_element_type=jnp.float32)
            + b_ref[...]
        )

    return pl.pallas_call(
        body,
        grid=(N // bn,),
        in_specs=[
            pl.BlockSpec((bn, D), lambda i: (i, 0)),
            pl.BlockSpec((D, D), lambda i: (0, 0)),
            pl.BlockSpec((1, D), lambda i: (0, 0)),
        ],
        out_specs=pl.BlockSpec((bn, D), lambda i: (i, 0)),
        out_shape=jax.ShapeDtypeStruct((N, D), jnp.float32),
    )(x, We, be.reshape(1, D))


def _mlp_layer(parts, h, W1, b1, W2, b2, bn):
    N, D = h.shape

    def body(p_ref, h_ref, w1_ref, b1_ref, w2_ref, b2_ref, o_ref):
        z = p_ref[0] + p_ref[1] + h_ref[...]
        z1 = jnp.maximum(
            jnp.dot(z, w1_ref[...], preferred_element_type=jnp.float32)
            + b1_ref[...],
            0.0,
        )
        z2 = (
            jnp.dot(z1, w2_ref[...], preferred_element_type=jnp.float32)
            + b2_ref[...]
        )
        o_ref[...] = jnp.maximum(z2, 0.0)

    return pl.pallas_call(
        body,
        grid=(N // bn,),
        in_specs=[
            pl.BlockSpec((NC, bn, D), lambda i: (0, i, 0)),
            pl.BlockSpec((bn, D), lambda i: (i, 0)),
            pl.BlockSpec((D, D), lambda i: (0, 0)),
            pl.BlockSpec((1, D), lambda i: (0, 0)),
            pl.BlockSpec((D, D), lambda i: (0, 0)),
            pl.BlockSpec((1, D), lambda i: (0, 0)),
        ],
        out_specs=pl.BlockSpec((bn, D), lambda i: (i, 0)),
        out_shape=jax.ShapeDtypeStruct((N, D), jnp.float32),
    )(parts, h, W1, b1.reshape(1, D), W2, b2.reshape(1, D))


def _pool_readout(h, batch3, rW1, rb1, rW2, rb2, G, bn):
    N, D = h.shape
    C = rb2.shape[0]
    nb = N // bn

    def body(h_ref, b_ref, w1_ref, b1_ref, w2_ref, b2_ref, o_ref, acc):
        i = pl.program_id(0)

        @pl.when(i == 0)
        def _():
            acc[...] = jnp.zeros_like(acc)

        ids = b_ref[0, 0, :]
        gi = lax.broadcasted_iota(jnp.int32, (G, bn), 0)
        mask = (ids[None, :] == gi).astype(jnp.float32)
        acc[...] += jnp.dot(mask, h_ref[...], preferred_element_type=jnp.float32)

        @pl.when(i == nb - 1)
        def _():
            p1 = jnp.maximum(
                jnp.dot(acc[...], w1_ref[...], preferred_element_type=jnp.float32)
                + b1_ref[...],
                0.0,
            )
            o_ref[...] = (
                jnp.dot(p1, w2_ref[...], preferred_element_type=jnp.float32)
                + b2_ref[...]
            )

    return pl.pallas_call(
        body,
        grid=(nb,),
        in_specs=[
            pl.BlockSpec((bn, D), lambda i: (i, 0)),
            pl.BlockSpec((1, 1, bn), lambda i: (i, 0, 0)),
            pl.BlockSpec((D, D), lambda i: (0, 0)),
            pl.BlockSpec((1, D), lambda i: (0, 0)),
            pl.BlockSpec((D, C), lambda i: (0, 0)),
            pl.BlockSpec((1, C), lambda i: (0, 0)),
        ],
        out_specs=pl.BlockSpec((G, C), lambda i: (0, 0)),
        out_shape=jax.ShapeDtypeStruct((G, C), jnp.float32),
        scratch_shapes=[pltpu.VMEM((G, D), jnp.float32)],
    )(h, batch3, rW1, rb1.reshape(1, D), rW2, rb2.reshape(1, C))


def kernel(x, c_2, u_2, batch, We, be, cW1, cb1, cW2, cb2, rW1, rb1, rW2, rb2):
    N, D = x.shape
    E = c_2.shape[0]
    L = cW1.shape[0]
    G = 64
    bn = 1000

    c2 = c_2.astype(jnp.int32)
    u2 = u_2.astype(jnp.int32)
    nchunk = -(-E // (NW * CH))
    nchunk += nchunk % 2
    ep = NW * nchunk * CH
    pad = ep - E
    c2p = jnp.concatenate([c2, jnp.zeros((pad,), jnp.int32)]).reshape(NW, nchunk, CH)
    acc_rows = ((N + NS * CH - 1) // (NS * CH)) * NS * CH
    trash = N + (jnp.arange(pad, dtype=jnp.int32) % (acc_rows - N))
    u2p = jnp.concatenate([u2, trash]).reshape(NW, nchunk, CH)

    h = _encoder(x, We, be, bn)
    for i in range(L):
        parts = _sc_aggregate(h, c2p, u2p, nchunk=nchunk)
        h = _mlp_layer(parts, h, cW1[i], cb1[i], cW2[i], cb2[i], bn)

    batch3 = batch.astype(jnp.int32).reshape(N // bn, 1, bn)
    return _pool_readout(h, batch3, rW1, rb1, rW2, rb2, G, bn)


# speedup vs baseline: 1.0841x; 1.0841x over previous
import functools
import jax
import jax.numpy as jnp
from jax import lax
from jax.experimental import pallas as pl
from jax.experimental.pallas import tpu as pltpu
from jax.experimental.pallas import tpu_sc as plsc

NC = 2
NS = 16
NW = NC * NS
CH = 128


@functools.partial(jax.jit, static_argnames=("nchunk",))
def _sc_aggregate(h, c2p, u2p, *, nchunk):
    N, D = h.shape
    acc_rows = ((N + NS * CH - 1) // (NS * CH)) * NS * CH
    zrows = acc_rows // NS

    assert nchunk % 4 == 0
    hs = nchunk // 2

    mesh = plsc.VectorSubcoreMesh(core_axis_name="c", subcore_axis_name="s")

    @functools.partial(
        pl.kernel,
        out_type=jax.ShapeDtypeStruct((NC, acc_rows, D), jnp.float32),
        mesh=mesh,
        scratch_types=[
            pltpu.VMEM((hs, CH), jnp.int32),
            pltpu.VMEM((hs, CH), jnp.int32),
            pltpu.VMEM((CH, D), jnp.float32),
            pltpu.VMEM((CH, D), jnp.float32),
            pltpu.VMEM_SHARED((acc_rows, D), jnp.float32),
            pltpu.SemaphoreType.DMA,
            pltpu.SemaphoreType.DMA,
        ],
    )
    def k(h_hbm, c2_hbm, u2_hbm, out_hbm, c2_v, u2_v, rows0_v, rows1_v,
          acc_s, gsem0, gsem1):
        rows_v = rows0_v
        cid = lax.axis_index("c")
        sid = lax.axis_index("s")
        wid = sid * NC + cid

        zeros16 = jnp.zeros((16,), jnp.float32)

        def zfill(i, carry):
            rows_v[i // (D // 16), pl.ds((i % (D // 16)) * 16, 16)] = zeros16
            return carry

        lax.fori_loop(0, CH * D // 16, zfill, 0)

        def zcopy(j, carry):
            pltpu.sync_copy(rows_v, acc_s.at[pl.ds(sid * zrows + j * CH, CH)])
            return carry

        lax.fori_loop(0, zrows // CH, zcopy, 0)
        plsc.subcore_barrier()

        def phase(p, carry):
            pltpu.sync_copy(c2_hbm.at[wid, pl.ds(p * hs, hs)], c2_v)
            pltpu.sync_copy(u2_hbm.at[wid, pl.ds(p * hs, hs)], u2_v)
            pltpu.async_copy(h_hbm.at[c2_v.at[0]], rows0_v, gsem0)

            def pair(g, carry2):
                j0 = 2 * g
                j1 = 2 * g + 1
                jn = jnp.minimum(2 * g + 2, hs - 1)
                pltpu.async_copy(h_hbm.at[c2_v.at[j1]], rows1_v, gsem1)
                pltpu.make_async_copy(
                    h_hbm.at[c2_v.at[j0]], rows0_v, gsem0).wait()
                pltpu.sync_copy(rows0_v, acc_s.at[u2_v.at[j0]], add=True)
                pltpu.async_copy(h_hbm.at[c2_v.at[jn]], rows0_v, gsem0)
                pltpu.make_async_copy(
                    h_hbm.at[c2_v.at[j1]], rows1_v, gsem1).wait()
                pltpu.sync_copy(rows1_v, acc_s.at[u2_v.at[j1]], add=True)
                return carry2

            lax.fori_loop(0, hs // 2, pair, 0)
            pltpu.make_async_copy(
                h_hbm.at[c2_v.at[hs - 1]], rows0_v, gsem0).wait()
            return carry

        lax.fori_loop(0, 2, phase, 0)
        plsc.subcore_barrier()

        pltpu.sync_copy(
            acc_s.at[pl.ds(sid * zrows, zrows)],
            out_hbm.at[cid, pl.ds(sid * zrows, zrows)],
        )

    return k(h, c2p, u2p)


def _encoder(x, We, be, bn):
    N, D = x.shape

    def body(x_ref, w_ref, b_ref, o_ref):
        o_ref[...] = (
            jnp.dot(x_ref[...], w_ref[...], preferred_element_type=jnp.float32)
            + b_ref[...]
        )

    return pl.pallas_call(
        body,
        grid=(N // bn,),
        in_specs=[
            pl.BlockSpec((bn, D), lambda i: (i, 0)),
            pl.BlockSpec((D, D), lambda i: (0, 0)),
            pl.BlockSpec((1, D), lambda i: (0, 0)),
        ],
        out_specs=pl.BlockSpec((bn, D), lambda i: (i, 0)),
        out_shape=jax.ShapeDtypeStruct((N, D), jnp.float32),
    )(x, We, be.reshape(1, D))


def _mlp_layer(parts, h, W1, b1, W2, b2, bn):
    N, D = h.shape

    def body(p_ref, h_ref, w1_ref, b1_ref, w2_ref, b2_ref, o_ref):
        z = p_ref[0] + p_ref[1] + h_ref[...]
        z1 = jnp.maximum(
            jnp.dot(z, w1_ref[...], preferred_element_type=jnp.float32)
            + b1_ref[...],
            0.0,
        )
        z2 = (
            jnp.dot(z1, w2_ref[...], preferred_element_type=jnp.float32)
            + b2_ref[...]
        )
        o_ref[...] = jnp.maximum(z2, 0.0)

    return pl.pallas_call(
        body,
        grid=(N // bn,),
        in_specs=[
            pl.BlockSpec((NC, bn, D), lambda i: (0, i, 0)),
            pl.BlockSpec((bn, D), lambda i: (i, 0)),
            pl.BlockSpec((D, D), lambda i: (0, 0)),
            pl.BlockSpec((1, D), lambda i: (0, 0)),
            pl.BlockSpec((D, D), lambda i: (0, 0)),
            pl.BlockSpec((1, D), lambda i: (0, 0)),
        ],
        out_specs=pl.BlockSpec((bn, D), lambda i: (i, 0)),
        out_shape=jax.ShapeDtypeStruct((N, D), jnp.float32),
    )(parts, h, W1, b1.reshape(1, D), W2, b2.reshape(1, D))


def _pool_readout(h, batch3, rW1, rb1, rW2, rb2, G, bn):
    N, D = h.shape
    C = rb2.shape[0]
    nb = N // bn

    def body(h_ref, b_ref, w1_ref, b1_ref, w2_ref, b2_ref, o_ref, acc):
        i = pl.program_id(0)

        @pl.when(i == 0)
        def _():
            acc[...] = jnp.zeros_like(acc)

        ids = b_ref[0, 0, :]
        gi = lax.broadcasted_iota(jnp.int32, (G, bn), 0)
        mask = (ids[None, :] == gi).astype(jnp.float32)
        acc[...] += jnp.dot(mask, h_ref[...], preferred_element_type=jnp.float32)

        @pl.when(i == nb - 1)
        def _():
            p1 = jnp.maximum(
                jnp.dot(acc[...], w1_ref[...], preferred_element_type=jnp.float32)
                + b1_ref[...],
                0.0,
            )
            o_ref[...] = (
                jnp.dot(p1, w2_ref[...], preferred_element_type=jnp.float32)
                + b2_ref[...]
            )

    return pl.pallas_call(
        body,
        grid=(nb,),
        in_specs=[
            pl.BlockSpec((bn, D), lambda i: (i, 0)),
            pl.BlockSpec((1, 1, bn), lambda i: (i, 0, 0)),
            pl.BlockSpec((D, D), lambda i: (0, 0)),
            pl.BlockSpec((1, D), lambda i: (0, 0)),
            pl.BlockSpec((D, C), lambda i: (0, 0)),
            pl.BlockSpec((1, C), lambda i: (0, 0)),
        ],
        out_specs=pl.BlockSpec((G, C), lambda i: (0, 0)),
        out_shape=jax.ShapeDtypeStruct((G, C), jnp.float32),
        scratch_shapes=[pltpu.VMEM((G, D), jnp.float32)],
    )(h, batch3, rW1, rb1.reshape(1, D), rW2, rb2.reshape(1, C))


def kernel(x, c_2, u_2, batch, We, be, cW1, cb1, cW2, cb2, rW1, rb1, rW2, rb2):
    N, D = x.shape
    E = c_2.shape[0]
    L = cW1.shape[0]
    G = 64
    bn = 1000

    c2 = c_2.astype(jnp.int32)
    u2 = u_2.astype(jnp.int32)
    nchunk = -(-E // (NW * CH))
    nchunk += (-nchunk) % 4
    ep = NW * nchunk * CH
    pad = ep - E
    c2p = jnp.concatenate([c2, jnp.zeros((pad,), jnp.int32)]).reshape(NW, nchunk, CH)
    acc_rows = ((N + NS * CH - 1) // (NS * CH)) * NS * CH
    trash = N + (jnp.arange(pad, dtype=jnp.int32) % (acc_rows - N))
    u2p = jnp.concatenate([u2, trash]).reshape(NW, nchunk, CH)

    h = _encoder(x, We, be, bn)
    for i in range(L):
        parts = _sc_aggregate(h, c2p, u2p, nchunk=nchunk)
        h = _mlp_layer(parts, h, cW1[i], cb1[i], cW2[i], cb2[i], bn)

    batch3 = batch.astype(jnp.int32).reshape(N // bn, 1, bn)
    return _pool_readout(h, batch3, rW1, rb1, rW2, rb2, G, bn)


# speedup vs baseline: 1.1653x; 1.0750x over previous
import functools
import jax
import jax.numpy as jnp
from jax import lax
from jax.experimental import pallas as pl
from jax.experimental.pallas import tpu as pltpu
from jax.experimental.pallas import tpu_sc as plsc

NC = 2
NS = 16
NW = NC * NS
CH = 128


@functools.partial(jax.jit, static_argnames=("nchunk",))
def _sc_aggregate(h, c2p, u2p, *, nchunk):
    N, D = h.shape
    acc_rows = ((N + NS * CH - 1) // (NS * CH)) * NS * CH
    zrows = acc_rows // NS

    assert nchunk % 4 == 0
    hs = nchunk // 2

    mesh = plsc.VectorSubcoreMesh(core_axis_name="c", subcore_axis_name="s")

    @functools.partial(
        pl.kernel,
        out_type=jax.ShapeDtypeStruct((NC, acc_rows, D), jnp.float32),
        mesh=mesh,
        scratch_types=[
            pltpu.VMEM((nchunk, CH), jnp.int32),
            pltpu.VMEM((nchunk, CH), jnp.int32),
            pltpu.VMEM((CH, D), jnp.float32),
            pltpu.VMEM_SHARED((acc_rows, D), jnp.float32),
            pltpu.SemaphoreType.DMA,
        ],
    )
    def k(h_hbm, c2_hbm, u2_hbm, out_hbm, c2_v, u2_v, rows_v, acc_s, gsem0):
        cid = lax.axis_index("c")
        sid = lax.axis_index("s")
        wid = sid * NC + cid

        zeros16 = jnp.zeros((16,), jnp.float32)

        def zfill(i, carry):
            rows_v[i // (D // 16), pl.ds((i % (D // 16)) * 16, 16)] = zeros16
            return carry

        lax.fori_loop(0, CH * D // 16, zfill, 0)

        def zcopy(j, carry):
            pltpu.sync_copy(rows_v, acc_s.at[pl.ds(sid * zrows + j * CH, CH)])
            return carry

        lax.fori_loop(0, zrows // CH, zcopy, 0)
        plsc.subcore_barrier()

        pltpu.sync_copy(c2_hbm.at[wid], c2_v)
        pltpu.sync_copy(u2_hbm.at[wid], u2_v)

        def chunk(j, carry):
            pltpu.async_copy(h_hbm.at[c2_v.at[j]], rows_v, gsem0).wait()
            pltpu.sync_copy(rows_v, acc_s.at[u2_v.at[j]], add=True)
            return carry

        lax.fori_loop(0, nchunk, chunk, 0)
        plsc.subcore_barrier()

        pltpu.sync_copy(
            acc_s.at[pl.ds(sid * zrows, zrows)],
            out_hbm.at[cid, pl.ds(sid * zrows, zrows)],
        )

    return k(h, c2p, u2p)


def _encoder(x, We, be, bn):
    N, D = x.shape

    def body(x_ref, w_ref, b_ref, o_ref):
        o_ref[...] = (
            jnp.dot(x_ref[...], w_ref[...], preferred_element_type=jnp.float32)
            + b_ref[...]
        )

    return pl.pallas_call(
        body,
        grid=(N // bn,),
        in_specs=[
            pl.BlockSpec((bn, D), lambda i: (i, 0)),
            pl.BlockSpec((D, D), lambda i: (0, 0)),
            pl.BlockSpec((1, D), lambda i: (0, 0)),
        ],
        out_specs=pl.BlockSpec((bn, D), lambda i: (i, 0)),
        out_shape=jax.ShapeDtypeStruct((N, D), jnp.float32),
    )(x, We, be.reshape(1, D))


def _mlp_layer(parts, h, W1, b1, W2, b2, bn):
    N, D = h.shape

    def body(p_ref, h_ref, w1_ref, b1_ref, w2_ref, b2_ref, o_ref):
        z = p_ref[0] + p_ref[1] + h_ref[...]
        z1 = jnp.maximum(
            jnp.dot(z, w1_ref[...], preferred_element_type=jnp.float32)
            + b1_ref[...],
            0.0,
        )
        z2 = (
            jnp.dot(z1, w2_ref[...], preferred_element_type=jnp.float32)
            + b2_ref[...]
        )
        o_ref[...] = jnp.maximum(z2, 0.0)

    return pl.pallas_call(
        body,
        grid=(N // bn,),
        in_specs=[
            pl.BlockSpec((NC, bn, D), lambda i: (0, i, 0)),
            pl.BlockSpec((bn, D), lambda i: (i, 0)),
            pl.BlockSpec((D, D), lambda i: (0, 0)),
            pl.BlockSpec((1, D), lambda i: (0, 0)),
            pl.BlockSpec((D, D), lambda i: (0, 0)),
            pl.BlockSpec((1, D), lambda i: (0, 0)),
        ],
        out_specs=pl.BlockSpec((bn, D), lambda i: (i, 0)),
        out_shape=jax.ShapeDtypeStruct((N, D), jnp.float32),
    )(parts, h, W1, b1.reshape(1, D), W2, b2.reshape(1, D))


def _pool_readout(h, batch3, rW1, rb1, rW2, rb2, G, bn):
    N, D = h.shape
    C = rb2.shape[0]
    nb = N // bn

    def body(h_ref, b_ref, w1_ref, b1_ref, w2_ref, b2_ref, o_ref, acc):
        i = pl.program_id(0)

        @pl.when(i == 0)
        def _():
            acc[...] = jnp.zeros_like(acc)

        ids = b_ref[0, 0, :]
        gi = lax.broadcasted_iota(jnp.int32, (G, bn), 0)
        mask = (ids[None, :] == gi).astype(jnp.float32)
        acc[...] += jnp.dot(mask, h_ref[...], preferred_element_type=jnp.float32)

        @pl.when(i == nb - 1)
        def _():
            p1 = jnp.maximum(
                jnp.dot(acc[...], w1_ref[...], preferred_element_type=jnp.float32)
                + b1_ref[...],
                0.0,
            )
            o_ref[...] = (
                jnp.dot(p1, w2_ref[...], preferred_element_type=jnp.float32)
                + b2_ref[...]
            )

    return pl.pallas_call(
        body,
        grid=(nb,),
        in_specs=[
            pl.BlockSpec((bn, D), lambda i: (i, 0)),
            pl.BlockSpec((1, 1, bn), lambda i: (i, 0, 0)),
            pl.BlockSpec((D, D), lambda i: (0, 0)),
            pl.BlockSpec((1, D), lambda i: (0, 0)),
            pl.BlockSpec((D, C), lambda i: (0, 0)),
            pl.BlockSpec((1, C), lambda i: (0, 0)),
        ],
        out_specs=pl.BlockSpec((G, C), lambda i: (0, 0)),
        out_shape=jax.ShapeDtypeStruct((G, C), jnp.float32),
        scratch_shapes=[pltpu.VMEM((G, D), jnp.float32)],
    )(h, batch3, rW1, rb1.reshape(1, D), rW2, rb2.reshape(1, C))


def kernel(x, c_2, u_2, batch, We, be, cW1, cb1, cW2, cb2, rW1, rb1, rW2, rb2):
    N, D = x.shape
    E = c_2.shape[0]
    L = cW1.shape[0]
    G = 64
    bn = 1000

    c2 = c_2.astype(jnp.int32)
    u2 = u_2.astype(jnp.int32)
    nchunk = -(-E // (NW * CH))
    nchunk += (-nchunk) % 4
    ep = NW * nchunk * CH
    pad = ep - E
    assert E % NW == 0 and pad % NW == 0
    acc_rows = ((N + NS * CH - 1) // (NS * CH)) * NS * CH
    pw = pad // NW
    trash = N + (jnp.arange(pw, dtype=jnp.int32) % (acc_rows - N))
    c2p = jnp.concatenate(
        [c2.reshape(NW, E // NW),
         jnp.zeros((NW, pw), jnp.int32)], axis=1).reshape(NW, nchunk, CH)
    u2p = jnp.concatenate(
        [u2.reshape(NW, E // NW),
         jnp.broadcast_to(trash, (NW, pw))], axis=1).reshape(NW, nchunk, CH)

    h = _encoder(x, We, be, bn)
    for i in range(L):
        parts = _sc_aggregate(h, c2p, u2p, nchunk=nchunk)
        h = _mlp_layer(parts, h, cW1[i], cb1[i], cW2[i], cb2[i], bn)

    batch3 = batch.astype(jnp.int32).reshape(N // bn, 1, bn)
    return _pool_readout(h, batch3, rW1, rb1, rW2, rb2, G, bn)


# speedup vs baseline: 1.1665x; 1.0010x over previous
import functools
import jax
import jax.numpy as jnp
from jax import lax
from jax.experimental import pallas as pl
from jax.experimental.pallas import tpu as pltpu
from jax.experimental.pallas import tpu_sc as plsc

NC = 2
NS = 16
NW = NC * NS
CH = 128


@functools.partial(jax.jit, static_argnames=("nchunk",))
def _sc_aggregate(h, c2p, u2p, *, nchunk):
    N, D = h.shape
    acc_rows = ((N + NS * CH - 1) // (NS * CH)) * NS * CH
    zrows = acc_rows // NS

    assert nchunk % 4 == 0
    hs = nchunk // 2

    mesh = plsc.VectorSubcoreMesh(core_axis_name="c", subcore_axis_name="s")

    @functools.partial(
        pl.kernel,
        out_type=jax.ShapeDtypeStruct((NC, acc_rows, D), jnp.float32),
        mesh=mesh,
        scratch_types=[
            pltpu.VMEM((nchunk, CH), jnp.int32),
            pltpu.VMEM((nchunk, CH), jnp.int32),
            pltpu.VMEM((CH, D), jnp.float32),
            pltpu.VMEM_SHARED((acc_rows, D), jnp.float32),
            pltpu.SemaphoreType.DMA,
        ],
    )
    def k(h_hbm, c2_hbm, u2_hbm, out_hbm, c2_v, u2_v, rows_v, acc_s, gsem0):
        cid = lax.axis_index("c")
        sid = lax.axis_index("s")
        wid = sid * NC + cid

        zeros16 = jnp.zeros((16,), jnp.float32)

        def zfill(i, carry):
            rows_v[i // (D // 16), pl.ds((i % (D // 16)) * 16, 16)] = zeros16
            return carry

        lax.fori_loop(0, CH * D // 16, zfill, 0)

        def zcopy(j, carry):
            pltpu.sync_copy(rows_v, acc_s.at[pl.ds(sid * zrows + j * CH, CH)])
            return carry

        lax.fori_loop(0, zrows // CH, zcopy, 0)
        plsc.subcore_barrier()

        pltpu.sync_copy(c2_hbm.at[wid], c2_v)
        pltpu.sync_copy(u2_hbm.at[wid], u2_v)

        def chunk(j, carry):
            pltpu.async_copy(h_hbm.at[c2_v.at[j]], rows_v, gsem0).wait()
            pltpu.sync_copy(rows_v, acc_s.at[u2_v.at[j]], add=True)
            return carry

        lax.fori_loop(0, nchunk, chunk, 0)
        plsc.subcore_barrier()

        pltpu.sync_copy(
            acc_s.at[pl.ds(sid * zrows, zrows)],
            out_hbm.at[cid, pl.ds(sid * zrows, zrows)],
        )

    return k(h, c2p, u2p)


def _encoder(x, We, be, bn):
    N, D = x.shape

    def body(x_ref, w_ref, b_ref, o_ref):
        o_ref[...] = (
            jnp.dot(x_ref[...], w_ref[...], preferred_element_type=jnp.float32)
            + b_ref[...]
        )

    return pl.pallas_call(
        body,
        grid=(N // bn,),
        in_specs=[
            pl.BlockSpec((bn, D), lambda i: (i, 0)),
            pl.BlockSpec((D, D), lambda i: (0, 0)),
            pl.BlockSpec((1, D), lambda i: (0, 0)),
        ],
        out_specs=pl.BlockSpec((bn, D), lambda i: (i, 0)),
        out_shape=jax.ShapeDtypeStruct((N, D), jnp.float32),
    )(x, We, be.reshape(1, D))


def _mlp_layer(parts, h, W1, b1, W2, b2, bn):
    N, D = h.shape

    def body(p_ref, h_ref, w1_ref, b1_ref, w2_ref, b2_ref, o_ref):
        z = p_ref[0] + p_ref[1] + h_ref[...]
        z1 = jnp.maximum(
            jnp.dot(z, w1_ref[...], preferred_element_type=jnp.float32)
            + b1_ref[...],
            0.0,
        )
        z2 = (
            jnp.dot(z1, w2_ref[...], preferred_element_type=jnp.float32)
            + b2_ref[...]
        )
        o_ref[...] = jnp.maximum(z2, 0.0)

    return pl.pallas_call(
        body,
        grid=(N // bn,),
        in_specs=[
            pl.BlockSpec((NC, bn, D), lambda i: (0, i, 0)),
            pl.BlockSpec((bn, D), lambda i: (i, 0)),
            pl.BlockSpec((D, D), lambda i: (0, 0)),
            pl.BlockSpec((1, D), lambda i: (0, 0)),
            pl.BlockSpec((D, D), lambda i: (0, 0)),
            pl.BlockSpec((1, D), lambda i: (0, 0)),
        ],
        out_specs=pl.BlockSpec((bn, D), lambda i: (i, 0)),
        out_shape=jax.ShapeDtypeStruct((N, D), jnp.float32),
    )(parts, h, W1, b1.reshape(1, D), W2, b2.reshape(1, D))


def _pool_readout(h, batch3, rW1, rb1, rW2, rb2, G, bn):
    N, D = h.shape
    C = rb2.shape[0]
    nb = N // bn

    def body(h_ref, b_ref, w1_ref, b1_ref, w2_ref, b2_ref, o_ref, acc):
        i = pl.program_id(0)

        @pl.when(i == 0)
        def _():
            acc[...] = jnp.zeros_like(acc)

        ids = b_ref[0, 0, :]
        gi = lax.broadcasted_iota(jnp.int32, (G, bn), 0)
        mask = (ids[None, :] == gi).astype(jnp.float32)
        acc[...] += jnp.dot(mask, h_ref[...], preferred_element_type=jnp.float32)

        @pl.when(i == nb - 1)
        def _():
            p1 = jnp.maximum(
                jnp.dot(acc[...], w1_ref[...], preferred_element_type=jnp.float32)
                + b1_ref[...],
                0.0,
            )
            o_ref[...] = (
                jnp.dot(p1, w2_ref[...], preferred_element_type=jnp.float32)
                + b2_ref[...]
            )

    return pl.pallas_call(
        body,
        grid=(nb,),
        in_specs=[
            pl.BlockSpec((bn, D), lambda i: (i, 0)),
            pl.BlockSpec((1, 1, bn), lambda i: (i, 0, 0)),
            pl.BlockSpec((D, D), lambda i: (0, 0)),
            pl.BlockSpec((1, D), lambda i: (0, 0)),
            pl.BlockSpec((D, C), lambda i: (0, 0)),
            pl.BlockSpec((1, C), lambda i: (0, 0)),
        ],
        out_specs=pl.BlockSpec((G, C), lambda i: (0, 0)),
        out_shape=jax.ShapeDtypeStruct((G, C), jnp.float32),
        scratch_shapes=[pltpu.VMEM((G, D), jnp.float32)],
    )(h, batch3, rW1, rb1.reshape(1, D), rW2, rb2.reshape(1, C))


def kernel(x, c_2, u_2, batch, We, be, cW1, cb1, cW2, cb2, rW1, rb1, rW2, rb2):
    N, D = x.shape
    E = c_2.shape[0]
    L = cW1.shape[0]
    G = 64
    bn = 1000

    c2 = c_2.astype(jnp.int32)
    u2 = u_2.astype(jnp.int32)
    nchunk = -(-E // (NW * CH))
    nchunk += (-nchunk) % 4
    ep = NW * nchunk * CH
    pad = ep - E
    assert E % NW == 0 and pad % NW == 0
    acc_rows = ((N + NS * CH - 1) // (NS * CH)) * NS * CH
    pw = pad // NW
    trash = N + (jnp.arange(NW, dtype=jnp.int32) // NC)[:, None]
    c2p = jnp.concatenate(
        [c2.reshape(NW, E // NW),
         jnp.zeros((NW, pw), jnp.int32)], axis=1).reshape(NW, nchunk, CH)
    u2p = jnp.concatenate(
        [u2.reshape(NW, E // NW),
         jnp.broadcast_to(trash, (NW, pw))], axis=1).reshape(NW, nchunk, CH)

    h = _encoder(x, We, be, bn)
    for i in range(L):
        parts = _sc_aggregate(h, c2p, u2p, nchunk=nchunk)
        h = _mlp_layer(parts, h, cW1[i], cb1[i], cW2[i], cb2[i], bn)

    batch3 = batch.astype(jnp.int32).reshape(N // bn, 1, bn)
    return _pool_readout(h, batch3, rW1, rb1, rW2, rb2, G, bn)


# speedup vs baseline: 1.5362x; 1.3169x over previous
import functools
import jax
import jax.numpy as jnp
from jax import lax
from jax.experimental import pallas as pl
from jax.experimental.pallas import tpu as pltpu
from jax.experimental.pallas import tpu_sc as plsc

NC = 2
NS = 16
NW = NC * NS
CH = 128


@functools.partial(jax.jit, static_argnames=("nchunk",))
def _sc_aggregate(h, c2p, u2p, *, nchunk):
    N, D = h.shape
    acc_rows = ((N + NS * CH - 1) // (NS * CH)) * NS * CH
    zrows = acc_rows // NS

    mesh = plsc.VectorSubcoreMesh(core_axis_name="c", subcore_axis_name="s")

    @functools.partial(
        pl.kernel,
        out_type=jax.ShapeDtypeStruct((NC, acc_rows, D), jnp.float32),
        mesh=mesh,
        scratch_types=[
            pltpu.VMEM((nchunk, CH), jnp.int32),
            pltpu.VMEM((nchunk, CH), jnp.int32),
            pltpu.VMEM((CH, D), jnp.float32),
            pltpu.VMEM_SHARED((acc_rows, D), jnp.float32),
            pltpu.SemaphoreType.DMA,
        ],
    )
    def k(h_hbm, c2_hbm, u2_hbm, out_hbm, c2_v, u2_v, rows_v, acc_s, gsem0):
        cid = lax.axis_index("c")
        sid = lax.axis_index("s")
        wid = sid * NC + cid

        pltpu.sync_copy(c2_hbm.at[wid], c2_v)
        pltpu.sync_copy(u2_hbm.at[wid], u2_v)

        zeros16 = jnp.zeros((16,), jnp.float32)

        def zfill(i, carry):
            rows_v[i // (D // 16), pl.ds((i % (D // 16)) * 16, 16)] = zeros16
            return carry

        lax.fori_loop(0, CH * D // 16, zfill, 0)

        def zcopy(j, carry):
            pltpu.sync_copy(rows_v, acc_s.at[pl.ds(sid * zrows + j * CH, CH)])
            return carry

        lax.fori_loop(0, zrows // CH, zcopy, 0)
        plsc.subcore_barrier()

        def chunk(j, carry):
            pltpu.async_copy(h_hbm.at[c2_v.at[j]], rows_v, gsem0).wait()
            pltpu.sync_copy(rows_v, acc_s.at[u2_v.at[j]], add=True)
            return carry

        lax.fori_loop(0, nchunk, chunk, 0)
        plsc.subcore_barrier()

        pltpu.sync_copy(
            acc_s.at[pl.ds(sid * zrows, zrows)],
            out_hbm.at[cid, pl.ds(sid * zrows, zrows)],
        )

    return k(h, c2p, u2p)


def _encoder(x, We, be, bn):
    N, D = x.shape

    def body(x_ref, w_ref, b_ref, o_ref):
        o_ref[...] = (
            jnp.dot(x_ref[...], w_ref[...], preferred_element_type=jnp.float32)
            + b_ref[...]
        )

    return pl.pallas_call(
        body,
        grid=(N // bn,),
        in_specs=[
            pl.BlockSpec((bn, D), lambda i: (i, 0)),
            pl.BlockSpec((D, D), lambda i: (0, 0)),
            pl.BlockSpec((1, D), lambda i: (0, 0)),
        ],
        out_specs=pl.BlockSpec((bn, D), lambda i: (i, 0)),
        out_shape=jax.ShapeDtypeStruct((N, D), jnp.float32),
    )(x, We, be.reshape(1, D))


def _mlp_layer(parts, h, W1, b1, W2, b2, bn):
    N, D = h.shape

    def body(p_ref, h_ref, w1_ref, b1_ref, w2_ref, b2_ref, o_ref):
        z = p_ref[0] + p_ref[1] + h_ref[...]
        z1 = jnp.maximum(
            jnp.dot(z, w1_ref[...], preferred_element_type=jnp.float32)
            + b1_ref[...],
            0.0,
        )
        z2 = (
            jnp.dot(z1, w2_ref[...], preferred_element_type=jnp.float32)
            + b2_ref[...]
        )
        o_ref[...] = jnp.maximum(z2, 0.0)

    return pl.pallas_call(
        body,
        grid=(N // bn,),
        in_specs=[
            pl.BlockSpec((NC, bn, D), lambda i: (0, i, 0)),
            pl.BlockSpec((bn, D), lambda i: (i, 0)),
            pl.BlockSpec((D, D), lambda i: (0, 0)),
            pl.BlockSpec((1, D), lambda i: (0, 0)),
            pl.BlockSpec((D, D), lambda i: (0, 0)),
            pl.BlockSpec((1, D), lambda i: (0, 0)),
        ],
        out_specs=pl.BlockSpec((bn, D), lambda i: (i, 0)),
        out_shape=jax.ShapeDtypeStruct((N, D), jnp.float32),
    )(parts, h, W1, b1.reshape(1, D), W2, b2.reshape(1, D))


def _pool_readout(h, batch3, rW1, rb1, rW2, rb2, G, bn):
    N, D = h.shape
    C = rb2.shape[0]
    nb = N // bn

    def body(h_ref, b_ref, w1_ref, b1_ref, w2_ref, b2_ref, o_ref, acc):
        i = pl.program_id(0)

        @pl.when(i == 0)
        def _():
            acc[...] = jnp.zeros_like(acc)

        ids = b_ref[0, 0, :]
        gi = lax.broadcasted_iota(jnp.int32, (G, bn), 0)
        mask = (ids[None, :] == gi).astype(jnp.float32)
        acc[...] += jnp.dot(mask, h_ref[...], preferred_element_type=jnp.float32)

        @pl.when(i == nb - 1)
        def _():
            p1 = jnp.maximum(
                jnp.dot(acc[...], w1_ref[...], preferred_element_type=jnp.float32)
                + b1_ref[...],
                0.0,
            )
            o_ref[...] = (
                jnp.dot(p1, w2_ref[...], preferred_element_type=jnp.float32)
                + b2_ref[...]
            )

    return pl.pallas_call(
        body,
        grid=(nb,),
        in_specs=[
            pl.BlockSpec((bn, D), lambda i: (i, 0)),
            pl.BlockSpec((1, 1, bn), lambda i: (i, 0, 0)),
            pl.BlockSpec((D, D), lambda i: (0, 0)),
            pl.BlockSpec((1, D), lambda i: (0, 0)),
            pl.BlockSpec((D, C), lambda i: (0, 0)),
            pl.BlockSpec((1, C), lambda i: (0, 0)),
        ],
        out_specs=pl.BlockSpec((G, C), lambda i: (0, 0)),
        out_shape=jax.ShapeDtypeStruct((G, C), jnp.float32),
        scratch_shapes=[pltpu.VMEM((G, D), jnp.float32)],
    )(h, batch3, rW1, rb1.reshape(1, D), rW2, rb2.reshape(1, C))


def kernel(x, c_2, u_2, batch, We, be, cW1, cb1, cW2, cb2, rW1, rb1, rW2, rb2):
    N, D = x.shape
    E = c_2.shape[0]
    L = cW1.shape[0]
    G = 64
    bn = 1000

    c2 = c_2.astype(jnp.int32)
    u2 = u_2.astype(jnp.int32)
    nchunk = -(-E // (NW * CH))
    ep = NW * nchunk * CH
    pad = ep - E
    c2p = jnp.concatenate([c2, jnp.zeros((pad,), jnp.int32)]).reshape(NW, nchunk, CH)
    u2p = jnp.concatenate([u2, jnp.full((pad,), N, jnp.int32)]).reshape(NW, nchunk, CH)

    h = _encoder(x, We, be, bn)
    for i in range(L):
        parts = _sc_aggregate(h, c2p, u2p, nchunk=nchunk)
        h = _mlp_layer(parts, h, cW1[i], cb1[i], cW2[i], cb2[i], bn)

    batch3 = batch.astype(jnp.int32).reshape(N // bn, 1, bn)
    return _pool_readout(h, batch3, rW1, rb1, rW2, rb2, G, bn)
